# baseline (speedup 1.0000x reference)
import functools

import jax
import jax.numpy as jnp
from jax import lax
from jax.experimental import pallas as pl
from jax.experimental.pallas import tpu as pltpu
from jax.experimental.pallas import tpu_sc as plsc

B = 16384
L_IDS = 20
D = 64
EPS = 1e-5

NUM_CORES = 2
NUM_SUBCORES = 16
NW = NUM_CORES * NUM_SUBCORES
B_PER_W = B // NW
CB = 32
N_CHUNKS = B_PER_W // CB
N_PAIRS = N_CHUNKS // 2
NV = D // 16


def _encoder_body(table_hbm, idxt_hbm, w_hbm, b_hbm, out_hbm,
                  idst_v, rows_v, out_v, wb_v, sems, out_sems):
    wid = lax.axis_index("s") * NUM_CORES + lax.axis_index("c")

    pltpu.sync_copy(w_hbm, wb_v.at[0])
    pltpu.sync_copy(b_hbm, wb_v.at[1])
    w_regs = [wb_v[0, pl.ds(k * 16, 16)] for k in range(NV)]
    b_regs = [wb_v[1, pl.ds(k * 16, 16)] for k in range(NV)]

    pltpu.sync_copy(idxt_hbm.at[:, pl.ds(wid * B_PER_W, B_PER_W)], idst_v)

    def gather_ops(c, buf):
        return [
            pltpu.make_async_copy(
                table_hbm.at[idst_v.at[l].at[pl.ds(c * CB, CB)]],
                rows_v.at[buf].at[l],
                sems.at[buf],
            )
            for l in range(L_IDS)
        ]

    def fire(c, buf):
        for cp in gather_ops(c, buf):
            cp.start()

    def wait(c, buf):
        for cp in gather_ops(c, buf):
            cp.wait()

    def out_store(c, buf):
        return pltpu.make_async_copy(
            out_v.at[buf],
            out_hbm.at[:, pl.ds(wid * B_PER_W + c * CB, CB)],
            out_sems.at[buf],
        )

    def compute_chunk(c, buf):
        @plsc.parallel_loop(0, CB, unroll=2)
        def batch_body(b):
            acc = [jnp.zeros((16,), jnp.float32) for _ in range(NV)]
            mr_sum = jnp.float32(0.0)
            for l in range(L_IDS):
                x = [rows_v[buf, l, b, pl.ds(k * 16, 16)] for k in range(NV)]
                t = (x[0] + x[1]) + (x[2] + x[3])
                q = (x[0] * x[0] + x[1] * x[1]) + (x[2] * x[2] + x[3] * x[3])
                s = jnp.sum(t)
                ssq = jnp.sum(q)
                mean = s * (1.0 / D)
                a = ssq * (1.0 / D) - mean * mean + EPS
                i = lax.bitcast_convert_type(a, jnp.int32)
                i = jnp.int32(0x5F3759DF) - (i >> 1)
                y = lax.bitcast_convert_type(i, jnp.float32)
                rinv = y * (1.5 - (0.5 * a) * (y * y))
                rb = jnp.full((16,), rinv, jnp.float32)
                acc = [acc[k] + x[k] * rb for k in range(NV)]
                mr_sum = mr_sum + mean * rinv
            mrv = jnp.full((16,), mr_sum, jnp.float32)
            col = jnp.full((16,), b, jnp.int32)
            for k in range(NV):
                val = (acc[k] - mrv) * (1.0 / L_IDS) * w_regs[k] + b_regs[k]
                row = lax.iota(jnp.int32, 16) + (k * 16)
                plsc.store_scatter(out_v.at[buf], [row, col], val)

    fire(0, 0)
    fire(1, 1)

    def half(p, c, buf):
        wait(c, buf)

        @pl.when(p > 0)
        def _():
            out_store(c - 2, buf).wait()

        compute_chunk(c, buf)
        out_store(c, buf).start()

        @pl.when(p < N_PAIRS - 1)
        def _():
            fire(c + 2, buf)

    def pair_body(p, _):
        c0 = p * 2
        half(p, c0, 0)
        half(p, c0 + 1, 1)
        return 0

    lax.fori_loop(0, N_PAIRS, pair_body, 0)
    out_store(N_CHUNKS - 2, 0).wait()
    out_store(N_CHUNKS - 1, 1).wait()


_encoder = functools.partial(
    pl.kernel,
    out_type=jax.ShapeDtypeStruct((D, B), jnp.float32),
    mesh=plsc.VectorSubcoreMesh(core_axis_name="c", subcore_axis_name="s"),
    compiler_params=pltpu.CompilerParams(
        needs_layout_passes=False, use_tc_tiling_on_sc=False
    ),
    scratch_types=[
        pltpu.VMEM((L_IDS, B_PER_W), jnp.int32),
        pltpu.VMEM((2, L_IDS, CB, D), jnp.float32),
        pltpu.VMEM((2, D, CB), jnp.float32),
        pltpu.VMEM((2, D), jnp.float32),
        pltpu.SemaphoreType.DMA((2,)),
        pltpu.SemaphoreType.DMA((2,)),
    ],
)(_encoder_body)


@jax.jit
def kernel(perturbation_ids, embedding_weight, ln_weight, ln_bias):
    ids_t = perturbation_ids.astype(jnp.int32).T
    return _encoder(embedding_weight, ids_t, ln_weight, ln_bias).T


# speedup vs baseline: 1.0321x; 1.0321x over previous
import functools

import jax
import jax.numpy as jnp
from jax import lax
from jax.experimental import pallas as pl
from jax.experimental.pallas import tpu as pltpu
from jax.experimental.pallas import tpu_sc as plsc

B = 16384
L_IDS = 20
D = 64
EPS = 1e-5

NUM_CORES = 2
NUM_SUBCORES = 16
NW = NUM_CORES * NUM_SUBCORES
B_PER_W = B // NW
CB = 32
N_CHUNKS = B_PER_W // CB
N_PAIRS = N_CHUNKS // 2
NV = D // 16


def _encoder_body(table_hbm, idxt_hbm, w_hbm, b_hbm, out_hbm,
                  idst_v, rows_v, out_v, wb_v, sems, out_sems):
    wid = lax.axis_index("s") * NUM_CORES + lax.axis_index("c")

    pltpu.sync_copy(w_hbm, wb_v.at[0])
    pltpu.sync_copy(b_hbm, wb_v.at[1])
    w_regs = [wb_v[0, pl.ds(k * 16, 16)] for k in range(NV)]
    b_regs = [wb_v[1, pl.ds(k * 16, 16)] for k in range(NV)]

    pltpu.sync_copy(idxt_hbm.at[:, pl.ds(wid * B_PER_W, B_PER_W)], idst_v)

    def gather_ops(c, buf):
        return [
            pltpu.make_async_copy(
                table_hbm.at[idst_v.at[l].at[pl.ds(c * CB, CB)]],
                rows_v.at[buf].at[l],
                sems.at[buf],
            )
            for l in range(L_IDS)
        ]

    def fire(c, buf):
        for cp in gather_ops(c, buf):
            cp.start()

    def wait(c, buf):
        for cp in gather_ops(c, buf):
            cp.wait()

    def out_store(c, buf):
        return pltpu.make_async_copy(
            out_v.at[buf],
            out_hbm.at[pl.ds(wid * B_PER_W + c * CB, CB)],
            out_sems.at[buf],
        )

    def compute_chunk(c, buf):
        @plsc.parallel_loop(0, CB, unroll=2)
        def batch_body(b):
            acc = [jnp.zeros((16,), jnp.float32) for _ in range(NV)]
            mr_sum = jnp.float32(0.0)
            for l in range(L_IDS):
                x = [rows_v[buf, l, b, pl.ds(k * 16, 16)] for k in range(NV)]
                t = (x[0] + x[1]) + (x[2] + x[3])
                q = (x[0] * x[0] + x[1] * x[1]) + (x[2] * x[2] + x[3] * x[3])
                s = jnp.sum(t)
                ssq = jnp.sum(q)
                mean = s * (1.0 / D)
                a = ssq * (1.0 / D) - mean * mean + EPS
                i = lax.bitcast_convert_type(a, jnp.int32)
                i = jnp.int32(0x5F3759DF) - (i >> 1)
                y = lax.bitcast_convert_type(i, jnp.float32)
                rinv = y * (1.5 - (0.5 * a) * (y * y))
                rb = jnp.full((16,), rinv, jnp.float32)
                acc = [acc[k] + x[k] * rb for k in range(NV)]
                mr_sum = mr_sum + mean * rinv
            mrv = jnp.full((16,), mr_sum, jnp.float32)
            for k in range(NV):
                out_v[buf, b, pl.ds(k * 16, 16)] = (
                    (acc[k] - mrv) * (1.0 / L_IDS) * w_regs[k] + b_regs[k]
                )

    fire(0, 0)
    fire(1, 1)

    def half(p, c, buf):
        wait(c, buf)

        @pl.when(p > 0)
        def _():
            out_store(c - 2, buf).wait()

        compute_chunk(c, buf)
        out_store(c, buf).start()

        @pl.when(p < N_PAIRS - 1)
        def _():
            fire(c + 2, buf)

    def pair_body(p, _):
        c0 = p * 2
        half(p, c0, 0)
        half(p, c0 + 1, 1)
        return 0

    lax.fori_loop(0, N_PAIRS, pair_body, 0)
    out_store(N_CHUNKS - 2, 0).wait()
    out_store(N_CHUNKS - 1, 1).wait()


_encoder = functools.partial(
    pl.kernel,
    out_type=jax.ShapeDtypeStruct((B, D), jnp.float32),
    mesh=plsc.VectorSubcoreMesh(core_axis_name="c", subcore_axis_name="s"),
    compiler_params=pltpu.CompilerParams(
        needs_layout_passes=False, use_tc_tiling_on_sc=False
    ),
    scratch_types=[
        pltpu.VMEM((L_IDS, B_PER_W), jnp.int32),
        pltpu.VMEM((2, L_IDS, CB, D), jnp.float32),
        pltpu.VMEM((2, CB, D), jnp.float32),
        pltpu.VMEM((2, D), jnp.float32),
        pltpu.SemaphoreType.DMA((2,)),
        pltpu.SemaphoreType.DMA((2,)),
    ],
)(_encoder_body)


@jax.jit
def kernel(perturbation_ids, embedding_weight, ln_weight, ln_bias):
    ids_t = perturbation_ids.astype(jnp.int32).T
    return _encoder(embedding_weight, ids_t, ln_weight, ln_bias)
